# baseline (speedup 1.0000x reference)
import functools

import jax
import jax.numpy as jnp
from jax import lax
from jax.experimental import pallas as pl
from jax.experimental.pallas import tpu as pltpu
from jax.experimental.pallas import tpu_sc as plsc

LANES = 16
CHUNK = 64
NC = 2
NS = 16
NW = NC * NS


def _cdiv(a, b):
    return (a + b - 1) // b


def _row_pieces(rows):
    pieces = []
    left = rows
    while left > 0:
        sz = min(CHUNK, left)
        pieces.append(sz)
        left -= sz
    return pieces


def _sc_mesh():
    return plsc.VectorSubcoreMesh(core_axis_name="c", subcore_axis_name="s")


@functools.lru_cache(maxsize=None)
def _make_deg_kernel(n_pad, cc2):
    rpt = n_pad // NS

    @functools.partial(
        pl.kernel,
        out_type=jax.ShapeDtypeStruct((NC, n_pad, LANES), jnp.float32),
        mesh=_sc_mesh(),
        compiler_params=pltpu.CompilerParams(use_tc_tiling_on_sc=False),
        scratch_types=[
            pltpu.VMEM((cc2, CHUNK), jnp.int32),
            pltpu.VMEM((cc2, CHUNK), jnp.int32),
            pltpu.VMEM((CHUNK, LANES), jnp.float32),
            pltpu.VMEM((CHUNK, LANES), jnp.float32),
            pltpu.VMEM((rpt, LANES), jnp.float32),
            pltpu.VMEM_SHARED((n_pad, LANES), jnp.float32),
        ],
    )
    def deg_kernel(src_hbm, dst_hbm, deg_hbm,
                   srcv, dstv, onesa_v, onesb_v, zv, deg_sh):
        cid = lax.axis_index("c")
        sid = lax.axis_index("s")
        wid = sid * NC + cid
        pltpu.sync_copy(src_hbm.at[wid], srcv)
        pltpu.sync_copy(dst_hbm.at[wid], dstv)

        lo = jnp.where(lax.iota(jnp.int32, LANES) < LANES // 2, 1.0, 0.0)
        hi = 1.0 - lo

        def _fill(i, _):
            onesa_v[i] = lo
            onesb_v[i] = hi
            return 0

        lax.fori_loop(0, CHUNK, _fill, 0)

        def _fillz(i, _):
            zv[i] = jnp.zeros((LANES,), jnp.float32)
            return 0

        lax.fori_loop(0, rpt, _fillz, 0)

        r0 = sid * rpt
        pltpu.sync_copy(zv, deg_sh.at[pl.ds(r0, rpt)])
        plsc.subcore_barrier()

        def _body(c, _):
            pltpu.sync_copy(onesa_v, deg_sh.at[srcv.at[c]], add=True)
            pltpu.sync_copy(onesb_v, deg_sh.at[dstv.at[c]], add=True)
            return 0

        lax.fori_loop(0, cc2, _body, 0)
        plsc.subcore_barrier()

        pltpu.sync_copy(deg_sh.at[pl.ds(r0, rpt)], zv)
        pltpu.sync_copy(zv, deg_hbm.at[cid, pl.ds(r0, rpt)])

    return deg_kernel


@functools.lru_cache(maxsize=None)
def _make_pass_kernel(n_pad, d, cc, cc2, nhalves):
    rpt = n_pad // NS
    pieces = _row_pieces(rpt)
    npairs = (cc + 2) // 2

    @functools.partial(
        pl.kernel,
        out_type=[jax.ShapeDtypeStruct((NC, n_pad, d), jnp.float32)
                  for _ in range(nhalves)],
        mesh=_sc_mesh(),
        compiler_params=pltpu.CompilerParams(use_tc_tiling_on_sc=False),
        scratch_types=[
            pltpu.VMEM((cc2, CHUNK), jnp.int32),
            pltpu.VMEM((cc2, CHUNK), jnp.int32),
            pltpu.VMEM((CHUNK, d), jnp.float32),
            pltpu.VMEM((CHUNK, d), jnp.float32),
            pltpu.VMEM_SHARED((n_pad, d), jnp.float32),
            pltpu.SemaphoreType.DMA,
            pltpu.SemaphoreType.DMA,
        ],
    )
    def pass_kernel(*args):
        tabs = args[:nhalves]
        src_hbm, dst_hbm = args[nhalves], args[nhalves + 1]
        outs = args[nhalves + 2:2 * nhalves + 2]
        srcv, dstv, bufa, bufb, acc_sh, sema, semb = args[2 * nhalves + 2:]
        cid = lax.axis_index("c")
        sid = lax.axis_index("s")
        wid = sid * NC + cid
        pltpu.sync_copy(src_hbm.at[wid], srcv)
        pltpu.sync_copy(dst_hbm.at[wid], dstv)
        r0 = sid * rpt

        for tab_hbm, out_hbm in zip(tabs, outs):
            def _fz(i, _):
                for k in range(d // LANES):
                    bufa[i, pl.ds(k * LANES, LANES)] = jnp.zeros(
                        (LANES,), jnp.float32)
                return 0

            lax.fori_loop(0, CHUNK, _fz, 0)
            off = 0
            for sz in pieces:
                pltpu.sync_copy(bufa.at[pl.ds(0, sz)],
                                acc_sh.at[pl.ds(r0 + off, sz)])
                off += sz
            plsc.subcore_barrier()

            pltpu.async_copy(tab_hbm.at[srcv.at[0]], bufa, sema)
            pltpu.async_copy(tab_hbm.at[srcv.at[1]], bufb, semb)

            def _body(i, _):
                c0 = 2 * i
                c1 = c0 + 1
                pltpu.make_async_copy(
                    tab_hbm.at[srcv.at[c0]], bufa, sema).wait()
                pltpu.sync_copy(bufa, acc_sh.at[dstv.at[c0]], add=True)
                pltpu.async_copy(tab_hbm.at[srcv.at[c0 + 2]], bufa, sema)
                pltpu.make_async_copy(
                    tab_hbm.at[srcv.at[c1]], bufb, semb).wait()
                pltpu.sync_copy(bufb, acc_sh.at[dstv.at[c1]], add=True)
                pltpu.async_copy(tab_hbm.at[srcv.at[c1 + 2]], bufb, semb)
                return 0

            lax.fori_loop(0, npairs, _body, 0)
            pltpu.make_async_copy(
                tab_hbm.at[srcv.at[cc + 2]], bufa, sema).wait()
            pltpu.make_async_copy(
                tab_hbm.at[srcv.at[cc + 3]], bufb, semb).wait()
            plsc.subcore_barrier()

            off = 0
            for sz in pieces:
                pltpu.sync_copy(acc_sh.at[pl.ds(r0 + off, sz)],
                                bufa.at[pl.ds(0, sz)])
                pltpu.sync_copy(bufa.at[pl.ds(0, sz)],
                                out_hbm.at[cid, pl.ds(r0 + off, sz)])
                off += sz

    return pass_kernel


def _tc_scale(x_pad, deg):
    n_pad, d = x_pad.shape
    d2 = d // 2

    def body(x_ref, dg_ref, o0_ref, o1_ref):
        do = dg_ref[0, :, 0:1] + dg_ref[1, :, 0:1]
        nsrc = lax.rsqrt(jnp.maximum(do, 1.0))
        v = x_ref[...] * nsrc
        o0_ref[...] = v[:, :d2]
        o1_ref[...] = v[:, d2:]

    return pl.pallas_call(
        body,
        out_shape=[jax.ShapeDtypeStruct((n_pad, d2), jnp.float32),
                   jax.ShapeDtypeStruct((n_pad, d2), jnp.float32)],
    )(x_pad, deg)


def _tc_mid(agg0, agg1, deg, w1, b1, w2):
    n_pad = agg0.shape[1]
    d2 = agg0.shape[2]
    c_dim = w2.shape[1]

    def body(a0_ref, a1_ref, dg_ref, w1_ref, b1_ref, w2_ref, o_ref):
        a0 = a0_ref[0] + a0_ref[1]
        a1 = a1_ref[0] + a1_ref[1]
        di = dg_ref[0, :, 8:9] + dg_ref[1, :, 8:9]
        nd = lax.rsqrt(jnp.maximum(di, 1.0))
        do = dg_ref[0, :, 0:1] + dg_ref[1, :, 0:1]
        nsrc = lax.rsqrt(jnp.maximum(do, 1.0))
        h1 = (jnp.dot(a0 * nd, w1_ref[0:d2, :],
                      preferred_element_type=jnp.float32)
              + jnp.dot(a1 * nd, w1_ref[d2:, :],
                        preferred_element_type=jnp.float32)
              + b1_ref[...])
        h1 = jnp.maximum(h1, 0.0)
        o_ref[...] = jnp.dot(h1 * nsrc, w2_ref[...],
                             preferred_element_type=jnp.float32)

    return pl.pallas_call(
        body,
        out_shape=jax.ShapeDtypeStruct((n_pad, c_dim), jnp.float32),
    )(agg0, agg1, deg, w1, b1, w2)


def _tc_final(agg, deg, b2):
    n_pad, c_dim = agg.shape[1], agg.shape[2]

    def body(agg_ref, dg_ref, b2_ref, o_ref):
        a = agg_ref[0] + agg_ref[1]
        di = dg_ref[0, :, 8:9] + dg_ref[1, :, 8:9]
        nd = lax.rsqrt(jnp.maximum(di, 1.0))
        o_ref[...] = a * nd + b2_ref[...]

    return pl.pallas_call(
        body,
        out_shape=jax.ShapeDtypeStruct((n_pad, c_dim), jnp.float32),
    )(agg, deg, b2)


def kernel(x, edge_index, W1, b1, W2, b2):
    n, d_in = x.shape
    e = edge_index.shape[1]

    n_pad = _cdiv(n + 1, 128) * 128

    cc = _cdiv(e, NW * CHUNK)
    cc += cc % 2
    cc2 = cc + 4
    e_pad = NW * cc * CHUNK

    src = edge_index[0].astype(jnp.int32)
    dst = edge_index[1].astype(jnp.int32)
    ndump = n_pad - n
    pad = n + (jnp.arange(e_pad - e, dtype=jnp.int32) % ndump)
    dummy = n + (jnp.arange(NW * (cc2 - cc) * CHUNK, dtype=jnp.int32)
                 % ndump).reshape(NW, cc2 - cc, CHUNK)
    src_a = jnp.concatenate(
        [jnp.concatenate([src, pad]).reshape(NW, cc, CHUNK), dummy], axis=1)
    dst_a = jnp.concatenate(
        [jnp.concatenate([dst, pad]).reshape(NW, cc, CHUNK), dummy], axis=1)

    x_pad = jnp.pad(x, ((0, n_pad - n), (0, 0)))

    deg = _make_deg_kernel(n_pad, cc2)(src_a, dst_a)
    h0, h1 = _tc_scale(x_pad, deg)
    agg0, agg1 = _make_pass_kernel(n_pad, d_in // 2, cc, cc2, 2)(
        h0, h1, src_a, dst_a)
    p = _tc_mid(agg0, agg1, deg, W1, b1.reshape(1, -1), W2)
    (agg2,) = _make_pass_kernel(n_pad, W2.shape[1], cc, cc2, 1)(
        p, src_a, dst_a)
    out_full = _tc_final(agg2, deg, b2.reshape(1, -1))
    return out_full[:n]


# speedup vs baseline: 1.2459x; 1.2459x over previous
import functools

import jax
import jax.numpy as jnp
from jax import lax
from jax.experimental import pallas as pl
from jax.experimental.pallas import tpu as pltpu
from jax.experimental.pallas import tpu_sc as plsc

LANES = 16
CHUNK = 128
NC = 2
NS = 16
NW = NC * NS


def _cdiv(a, b):
    return (a + b - 1) // b


def _row_pieces(rows):
    pieces = []
    left = rows
    while left > 0:
        sz = min(CHUNK, left)
        pieces.append(sz)
        left -= sz
    return pieces


def _sc_mesh():
    return plsc.VectorSubcoreMesh(core_axis_name="c", subcore_axis_name="s")


@functools.lru_cache(maxsize=None)
def _make_deg_kernel(n_pad, cc2):
    rpt = n_pad // NS

    @functools.partial(
        pl.kernel,
        out_type=jax.ShapeDtypeStruct((NC, n_pad, LANES), jnp.float32),
        mesh=_sc_mesh(),
        compiler_params=pltpu.CompilerParams(use_tc_tiling_on_sc=False),
        scratch_types=[
            pltpu.VMEM((cc2, CHUNK), jnp.int32),
            pltpu.VMEM((cc2, CHUNK), jnp.int32),
            pltpu.VMEM((CHUNK, LANES), jnp.float32),
            pltpu.VMEM((CHUNK, LANES), jnp.float32),
            pltpu.VMEM((rpt, LANES), jnp.float32),
            pltpu.VMEM_SHARED((n_pad, LANES), jnp.float32),
        ],
    )
    def deg_kernel(src_hbm, dst_hbm, deg_hbm,
                   srcv, dstv, onesa_v, onesb_v, zv, deg_sh):
        cid = lax.axis_index("c")
        sid = lax.axis_index("s")
        wid = sid * NC + cid
        pltpu.sync_copy(src_hbm.at[wid], srcv)
        pltpu.sync_copy(dst_hbm.at[wid], dstv)

        lo = jnp.where(lax.iota(jnp.int32, LANES) < LANES // 2, 1.0, 0.0)
        hi = 1.0 - lo

        def _fill(i, _):
            onesa_v[i] = lo
            onesb_v[i] = hi
            return 0

        lax.fori_loop(0, CHUNK, _fill, 0)

        def _fillz(i, _):
            zv[i] = jnp.zeros((LANES,), jnp.float32)
            return 0

        lax.fori_loop(0, rpt, _fillz, 0)

        r0 = sid * rpt
        pltpu.sync_copy(zv, deg_sh.at[pl.ds(r0, rpt)])
        plsc.subcore_barrier()

        def _body(c, _):
            pltpu.sync_copy(onesa_v, deg_sh.at[srcv.at[c]], add=True)
            pltpu.sync_copy(onesb_v, deg_sh.at[dstv.at[c]], add=True)
            return 0

        lax.fori_loop(0, cc2, _body, 0)
        plsc.subcore_barrier()

        pltpu.sync_copy(deg_sh.at[pl.ds(r0, rpt)], zv)
        pltpu.sync_copy(zv, deg_hbm.at[cid, pl.ds(r0, rpt)])

    return deg_kernel


@functools.lru_cache(maxsize=None)
def _make_pass_kernel(n_pad, d, cc, cc2, nhalves):
    rpt = n_pad // NS
    pieces = _row_pieces(rpt)
    npairs = (cc + 2) // 2

    @functools.partial(
        pl.kernel,
        out_type=[jax.ShapeDtypeStruct((NC, n_pad, d), jnp.float32)
                  for _ in range(nhalves)],
        mesh=_sc_mesh(),
        compiler_params=pltpu.CompilerParams(use_tc_tiling_on_sc=False),
        scratch_types=[
            pltpu.VMEM((cc2, CHUNK), jnp.int32),
            pltpu.VMEM((cc2, CHUNK), jnp.int32),
            pltpu.VMEM((CHUNK, d), jnp.float32),
            pltpu.VMEM((CHUNK, d), jnp.float32),
            pltpu.VMEM_SHARED((n_pad, d), jnp.float32),
            pltpu.SemaphoreType.DMA,
            pltpu.SemaphoreType.DMA,
        ],
    )
    def pass_kernel(*args):
        tabs = args[:nhalves]
        src_hbm, dst_hbm = args[nhalves], args[nhalves + 1]
        outs = args[nhalves + 2:2 * nhalves + 2]
        srcv, dstv, bufa, bufb, acc_sh, sema, semb = args[2 * nhalves + 2:]
        cid = lax.axis_index("c")
        sid = lax.axis_index("s")
        wid = sid * NC + cid
        pltpu.sync_copy(src_hbm.at[wid], srcv)
        pltpu.sync_copy(dst_hbm.at[wid], dstv)
        r0 = sid * rpt

        for tab_hbm, out_hbm in zip(tabs, outs):
            def _fz(i, _):
                for k in range(d // LANES):
                    bufa[i, pl.ds(k * LANES, LANES)] = jnp.zeros(
                        (LANES,), jnp.float32)
                return 0

            lax.fori_loop(0, CHUNK, _fz, 0)
            off = 0
            for sz in pieces:
                pltpu.sync_copy(bufa.at[pl.ds(0, sz)],
                                acc_sh.at[pl.ds(r0 + off, sz)])
                off += sz
            plsc.subcore_barrier()

            pltpu.async_copy(tab_hbm.at[srcv.at[0]], bufa, sema)
            pltpu.async_copy(tab_hbm.at[srcv.at[1]], bufb, semb)

            def _body(i, _):
                c0 = 2 * i
                c1 = c0 + 1
                pltpu.make_async_copy(
                    tab_hbm.at[srcv.at[c0]], bufa, sema).wait()
                pltpu.sync_copy(bufa, acc_sh.at[dstv.at[c0]], add=True)
                pltpu.async_copy(tab_hbm.at[srcv.at[c0 + 2]], bufa, sema)
                pltpu.make_async_copy(
                    tab_hbm.at[srcv.at[c1]], bufb, semb).wait()
                pltpu.sync_copy(bufb, acc_sh.at[dstv.at[c1]], add=True)
                pltpu.async_copy(tab_hbm.at[srcv.at[c1 + 2]], bufb, semb)
                return 0

            lax.fori_loop(0, npairs, _body, 0)
            pltpu.make_async_copy(
                tab_hbm.at[srcv.at[cc + 2]], bufa, sema).wait()
            pltpu.make_async_copy(
                tab_hbm.at[srcv.at[cc + 3]], bufb, semb).wait()
            plsc.subcore_barrier()

            off = 0
            for sz in pieces:
                pltpu.sync_copy(acc_sh.at[pl.ds(r0 + off, sz)],
                                bufa.at[pl.ds(0, sz)])
                pltpu.sync_copy(bufa.at[pl.ds(0, sz)],
                                out_hbm.at[cid, pl.ds(r0 + off, sz)])
                off += sz

    return pass_kernel


def _tc_scale(x_pad, deg):
    n_pad, d = x_pad.shape
    d2 = d // 2

    def body(x_ref, dg_ref, o0_ref, o1_ref):
        do = dg_ref[0, :, 0:1] + dg_ref[1, :, 0:1]
        nsrc = lax.rsqrt(jnp.maximum(do, 1.0))
        v = x_ref[...] * nsrc
        o0_ref[...] = v[:, :d2]
        o1_ref[...] = v[:, d2:]

    return pl.pallas_call(
        body,
        out_shape=[jax.ShapeDtypeStruct((n_pad, d2), jnp.float32),
                   jax.ShapeDtypeStruct((n_pad, d2), jnp.float32)],
    )(x_pad, deg)


def _tc_mid(agg0, agg1, deg, w1, b1, w2):
    n_pad = agg0.shape[1]
    d2 = agg0.shape[2]
    c_dim = w2.shape[1]

    def body(a0_ref, a1_ref, dg_ref, w1_ref, b1_ref, w2_ref, o_ref):
        a0 = a0_ref[0] + a0_ref[1]
        a1 = a1_ref[0] + a1_ref[1]
        di = dg_ref[0, :, 8:9] + dg_ref[1, :, 8:9]
        nd = lax.rsqrt(jnp.maximum(di, 1.0))
        do = dg_ref[0, :, 0:1] + dg_ref[1, :, 0:1]
        nsrc = lax.rsqrt(jnp.maximum(do, 1.0))
        h1 = (jnp.dot(a0 * nd, w1_ref[0:d2, :],
                      preferred_element_type=jnp.float32)
              + jnp.dot(a1 * nd, w1_ref[d2:, :],
                        preferred_element_type=jnp.float32)
              + b1_ref[...])
        h1 = jnp.maximum(h1, 0.0)
        o_ref[...] = jnp.dot(h1 * nsrc, w2_ref[...],
                             preferred_element_type=jnp.float32)

    return pl.pallas_call(
        body,
        out_shape=jax.ShapeDtypeStruct((n_pad, c_dim), jnp.float32),
    )(agg0, agg1, deg, w1, b1, w2)


def _tc_final(agg, deg, b2):
    n_pad, c_dim = agg.shape[1], agg.shape[2]

    def body(agg_ref, dg_ref, b2_ref, o_ref):
        a = agg_ref[0] + agg_ref[1]
        di = dg_ref[0, :, 8:9] + dg_ref[1, :, 8:9]
        nd = lax.rsqrt(jnp.maximum(di, 1.0))
        o_ref[...] = a * nd + b2_ref[...]

    return pl.pallas_call(
        body,
        out_shape=jax.ShapeDtypeStruct((n_pad, c_dim), jnp.float32),
    )(agg, deg, b2)


def kernel(x, edge_index, W1, b1, W2, b2):
    n, d_in = x.shape
    e = edge_index.shape[1]

    n_pad = _cdiv(n + 1, 128) * 128

    cc = _cdiv(e, NW * CHUNK)
    cc += cc % 2
    cc2 = cc + 4
    e_pad = NW * cc * CHUNK

    src = edge_index[0].astype(jnp.int32)
    dst = edge_index[1].astype(jnp.int32)
    ndump = n_pad - n
    pad = n + (jnp.arange(e_pad - e, dtype=jnp.int32) % ndump)
    dummy = n + (jnp.arange(NW * (cc2 - cc) * CHUNK, dtype=jnp.int32)
                 % ndump).reshape(NW, cc2 - cc, CHUNK)
    src_a = jnp.concatenate(
        [jnp.concatenate([src, pad]).reshape(NW, cc, CHUNK), dummy], axis=1)
    dst_a = jnp.concatenate(
        [jnp.concatenate([dst, pad]).reshape(NW, cc, CHUNK), dummy], axis=1)

    x_pad = jnp.pad(x, ((0, n_pad - n), (0, 0)))

    deg = _make_deg_kernel(n_pad, cc2)(src_a, dst_a)
    h0, h1 = _tc_scale(x_pad, deg)
    agg0, agg1 = _make_pass_kernel(n_pad, d_in // 2, cc, cc2, 2)(
        h0, h1, src_a, dst_a)
    p = _tc_mid(agg0, agg1, deg, W1, b1.reshape(1, -1), W2)
    (agg2,) = _make_pass_kernel(n_pad, W2.shape[1], cc, cc2, 1)(
        p, src_a, dst_a)
    out_full = _tc_final(agg2, deg, b2.reshape(1, -1))
    return out_full[:n]
